# Initial kernel scaffold; baseline (speedup 1.0000x reference)
import functools

import jax
import jax.numpy as jnp
from jax import lax
from jax.experimental import pallas as pl
from jax.experimental.pallas import tpu as pltpu
from jax.experimental.pallas import tpu_sc as plsc

B, T = 1024, 200
NUM_CB, CB_SIZE, EMB, HID = 32, 1024, 64, 128
N = B * T
NW = 32
P = N // NW
C = 32
ROWS = C * NUM_CB
NCHUNK = P // C


def _sc_embed_body(idx_hbm, tbl_hbm, out_hbm, idx_v, rows_v, out_v, sem):
    wid = lax.axis_index("s") * 2 + lax.axis_index("c")

    def chunk_body(g, carry):
        pos0 = wid * P + g * C
        row8 = (pos0 * NUM_CB) // 128
        pltpu.sync_copy(idx_hbm.at[pl.ds(row8, ROWS // 128)], idx_v)
        copies = []
        for i in range(ROWS // 128):
            copies.append(
                pltpu.async_copy(
                    tbl_hbm.at[idx_v.at[i]],
                    rows_v.at[pl.ds(i * 128, 128)],
                    sem,
                )
            )
        for cp in copies:
            cp.wait()

        def pos_body(j, carry2):
            base = j * NUM_CB
            for k in range(EMB // 16):
                sl = pl.ds(k * 16, 16)
                acc = rows_v[base, sl]
                for r in range(1, NUM_CB):
                    acc = acc + rows_v[base + r, sl]
                out_v[j, sl] = acc
            return carry2

        lax.fori_loop(0, C, pos_body, 0, unroll=False)
        pltpu.sync_copy(out_v, out_hbm.at[pl.ds(pos0, C)])
        return carry

    lax.fori_loop(0, NCHUNK, chunk_body, 0, unroll=False)


def _sc_embed(idx2, table):
    mesh = plsc.VectorSubcoreMesh(core_axis_name="c", subcore_axis_name="s")
    return pl.kernel(
        _sc_embed_body,
        mesh=mesh,
        out_type=jax.ShapeDtypeStruct((N, EMB), jnp.float32),
        scratch_types=[
            pltpu.VMEM((ROWS // 128, 128), jnp.int32),
            pltpu.VMEM((ROWS, EMB), jnp.float32),
            pltpu.VMEM((C, EMB), jnp.float32),
            pltpu.SemaphoreType.DMA,
        ],
    )(idx2, table)


def _tc_body(emb_ref, m0_ref, b0_ref, whh0_ref, wih1_ref, b1_ref, whh1_ref,
             fcw_ref, fcb_ref, out_ref, h0, c0, h1, c1):
    t = pl.program_id(0)

    @pl.when(t == 0)
    def _():
        h0[...] = jnp.zeros_like(h0)
        c0[...] = jnp.zeros_like(c0)
        h1[...] = jnp.zeros_like(h1)
        c1[...] = jnp.zeros_like(c1)

    x = emb_ref[0]
    g0 = (jnp.dot(x, m0_ref[...], preferred_element_type=jnp.float32)
          + b0_ref[0]
          + jnp.dot(h0[...], whh0_ref[...], preferred_element_type=jnp.float32))
    i0 = jax.nn.sigmoid(g0[:, :HID])
    f0 = jax.nn.sigmoid(g0[:, HID:2 * HID])
    gg0 = jnp.tanh(g0[:, 2 * HID:3 * HID])
    o0 = jax.nn.sigmoid(g0[:, 3 * HID:])
    c0n = f0 * c0[...] + i0 * gg0
    h0n = o0 * jnp.tanh(c0n)
    c0[...] = c0n
    h0[...] = h0n

    g1 = (jnp.dot(h0n, wih1_ref[...], preferred_element_type=jnp.float32)
          + b1_ref[0]
          + jnp.dot(h1[...], whh1_ref[...], preferred_element_type=jnp.float32))
    i1 = jax.nn.sigmoid(g1[:, :HID])
    f1 = jax.nn.sigmoid(g1[:, HID:2 * HID])
    gg1 = jnp.tanh(g1[:, 2 * HID:3 * HID])
    o1 = jax.nn.sigmoid(g1[:, 3 * HID:])
    c1n = f1 * c1[...] + i1 * gg1
    h1n = o1 * jnp.tanh(c1n)
    c1[...] = c1n
    h1[...] = h1n

    out_ref[0] = jnp.dot(h1n, fcw_ref[...], preferred_element_type=jnp.float32) + fcb_ref[0]


def _tc_lstm(emb3, m0, b0, whh0, wih1, b1, whh1, fcw, fcb):
    full = lambda shape: pl.BlockSpec(shape, lambda t: (0,) * len(shape))
    return pl.pallas_call(
        _tc_body,
        grid=(T,),
        in_specs=[
            pl.BlockSpec((1, B, EMB), lambda t: (t, 0, 0)),
            full((EMB, 4 * HID)),
            full((1, 4 * HID)),
            full((HID, 4 * HID)),
            full((HID, 4 * HID)),
            full((1, 4 * HID)),
            full((HID, 4 * HID)),
            full((HID, 24)),
            full((1, 24)),
        ],
        out_specs=pl.BlockSpec((1, B, 24), lambda t: (t, 0, 0)),
        out_shape=jax.ShapeDtypeStruct((T, B, 24), jnp.float32),
        scratch_shapes=[pltpu.VMEM((B, HID), jnp.float32) for _ in range(4)],
        compiler_params=pltpu.CompilerParams(
            dimension_semantics=("arbitrary",)),
    )(emb3, m0, b0, whh0, wih1, b1, whh1, fcw, fcb)


def kernel(x, emb_tables, proj_W, proj_b, l0_W_ih, l0_W_hh, l0_b_ih, l0_b_hh,
           l1_W_ih, l1_W_hh, l1_b_ih, l1_b_hh, fc_W, fc_b):
    offs = (jnp.arange(NUM_CB, dtype=jnp.int32) * CB_SIZE)
    idx = x.transpose(1, 0, 2) + offs
    idx2 = idx.reshape(-1, 128)
    table = emb_tables.reshape(NUM_CB * CB_SIZE, EMB)

    emb = _sc_embed(idx2, table)
    emb3 = emb.reshape(T, B, EMB)

    m0 = proj_W.T @ l0_W_ih.T
    b0 = (proj_b @ l0_W_ih.T + l0_b_ih + l0_b_hh).reshape(1, 4 * HID)
    whh0 = l0_W_hh.T
    wih1 = l1_W_ih.T
    b1 = (l1_b_ih + l1_b_hh).reshape(1, 4 * HID)
    whh1 = l1_W_hh.T
    fcw = fc_W.reshape(24, HID).T
    fcb = fc_b.reshape(1, 24)

    outT = _tc_lstm(emb3, m0, b0, whh0, wih1, b1, whh1, fcw, fcb)
    return outT.transpose(1, 0, 2).reshape(B, T, 4, 6)


# speedup vs baseline: 42.1520x; 42.1520x over previous
import functools

import jax
import jax.numpy as jnp
from jax import lax
from jax.experimental import pallas as pl
from jax.experimental.pallas import tpu as pltpu
from jax.experimental.pallas import tpu_sc as plsc

B, T = 1024, 200
NUM_CB, CB_SIZE, EMB, HID = 32, 1024, 64, 128
N = B * T
NW = 32
P = N // NW
C = 32
ROWS = C * NUM_CB
NCHUNK = P // C


def _sc_embed_body(idx_hbm, tbl_hbm, out_hbm, idx_v, rows_v, out_v, sem):
    wid = lax.axis_index("s") * 2 + lax.axis_index("c")

    def chunk_body(g, carry):
        pos0 = pl.multiple_of(wid * P + g * C, C)
        row8 = pl.multiple_of((pos0 * NUM_CB) // 128, 8)
        pltpu.sync_copy(idx_hbm.at[pl.ds(row8, ROWS // 128)], idx_v)
        copies = []
        for i in range(ROWS // 128):
            copies.append(
                pltpu.async_copy(
                    tbl_hbm.at[idx_v.at[i]],
                    rows_v.at[pl.ds(i * 128, 128)],
                    sem,
                )
            )
        for cp in copies:
            cp.wait()

        def pos_body(j, carry2):
            base = j * NUM_CB
            for k in range(EMB // 16):
                sl = pl.ds(k * 16, 16)
                acc = rows_v[base, sl]
                for r in range(1, NUM_CB):
                    acc = acc + rows_v[base + r, sl]
                out_v[j, sl] = acc
            return carry2

        lax.fori_loop(0, C, pos_body, 0, unroll=False)
        pltpu.sync_copy(out_v, out_hbm.at[pl.ds(pos0, C)])
        return carry

    lax.fori_loop(0, NCHUNK, chunk_body, 0, unroll=False)


def _sc_embed(idx2, table):
    mesh = plsc.VectorSubcoreMesh(core_axis_name="c", subcore_axis_name="s")
    return pl.kernel(
        _sc_embed_body,
        mesh=mesh,
        out_type=jax.ShapeDtypeStruct((N, EMB), jnp.float32),
        scratch_types=[
            pltpu.VMEM((ROWS // 128, 128), jnp.int32),
            pltpu.VMEM((ROWS, EMB), jnp.float32),
            pltpu.VMEM((C, EMB), jnp.float32),
            pltpu.SemaphoreType.DMA,
        ],
        compiler_params=pltpu.CompilerParams(use_tc_tiling_on_sc=False),
    )(idx2, table)


def _tc_body(emb_ref, m0_ref, b0_ref, whh0_ref, wih1_ref, b1_ref, whh1_ref,
             fcw_ref, fcb_ref, out_ref, h0, c0, h1, c1):
    t = pl.program_id(0)

    @pl.when(t == 0)
    def _():
        h0[...] = jnp.zeros_like(h0)
        c0[...] = jnp.zeros_like(c0)
        h1[...] = jnp.zeros_like(h1)
        c1[...] = jnp.zeros_like(c1)

    x = emb_ref[0]
    g0 = (jnp.dot(x, m0_ref[...], preferred_element_type=jnp.float32)
          + b0_ref[0]
          + jnp.dot(h0[...], whh0_ref[...], preferred_element_type=jnp.float32))
    i0 = jax.nn.sigmoid(g0[:, :HID])
    f0 = jax.nn.sigmoid(g0[:, HID:2 * HID])
    gg0 = jnp.tanh(g0[:, 2 * HID:3 * HID])
    o0 = jax.nn.sigmoid(g0[:, 3 * HID:])
    c0n = f0 * c0[...] + i0 * gg0
    h0n = o0 * jnp.tanh(c0n)
    c0[...] = c0n
    h0[...] = h0n

    g1 = (jnp.dot(h0n, wih1_ref[...], preferred_element_type=jnp.float32)
          + b1_ref[0]
          + jnp.dot(h1[...], whh1_ref[...], preferred_element_type=jnp.float32))
    i1 = jax.nn.sigmoid(g1[:, :HID])
    f1 = jax.nn.sigmoid(g1[:, HID:2 * HID])
    gg1 = jnp.tanh(g1[:, 2 * HID:3 * HID])
    o1 = jax.nn.sigmoid(g1[:, 3 * HID:])
    c1n = f1 * c1[...] + i1 * gg1
    h1n = o1 * jnp.tanh(c1n)
    c1[...] = c1n
    h1[...] = h1n

    out_ref[0] = jnp.dot(h1n, fcw_ref[...], preferred_element_type=jnp.float32) + fcb_ref[0]


def _tc_lstm(emb3, m0, b0, whh0, wih1, b1, whh1, fcw, fcb):
    full = lambda shape: pl.BlockSpec(shape, lambda t: (0,) * len(shape))
    return pl.pallas_call(
        _tc_body,
        grid=(T,),
        in_specs=[
            pl.BlockSpec((1, B, EMB), lambda t: (t, 0, 0)),
            full((EMB, 4 * HID)),
            full((1, 4 * HID)),
            full((HID, 4 * HID)),
            full((HID, 4 * HID)),
            full((1, 4 * HID)),
            full((HID, 4 * HID)),
            full((HID, 24)),
            full((1, 24)),
        ],
        out_specs=pl.BlockSpec((1, B, 24), lambda t: (t, 0, 0)),
        out_shape=jax.ShapeDtypeStruct((T, B, 24), jnp.float32),
        scratch_shapes=[pltpu.VMEM((B, HID), jnp.float32) for _ in range(4)],
        compiler_params=pltpu.CompilerParams(
            dimension_semantics=("arbitrary",)),
    )(emb3, m0, b0, whh0, wih1, b1, whh1, fcw, fcb)


def kernel(x, emb_tables, proj_W, proj_b, l0_W_ih, l0_W_hh, l0_b_ih, l0_b_hh,
           l1_W_ih, l1_W_hh, l1_b_ih, l1_b_hh, fc_W, fc_b):
    offs = (jnp.arange(NUM_CB, dtype=jnp.int32) * CB_SIZE)
    idx = x.transpose(1, 0, 2) + offs
    idx2 = idx.reshape(-1, 128)
    table = emb_tables.reshape(NUM_CB * CB_SIZE, EMB)

    emb = _sc_embed(idx2, table)
    emb3 = emb.reshape(T, B, EMB)

    m0 = proj_W.T @ l0_W_ih.T
    b0 = (proj_b @ l0_W_ih.T + l0_b_ih + l0_b_hh).reshape(1, 4 * HID)
    whh0 = l0_W_hh.T
    wih1 = l1_W_ih.T
    b1 = (l1_b_ih + l1_b_hh).reshape(1, 4 * HID)
    whh1 = l1_W_hh.T
    fcw = fc_W.reshape(24, HID).T
    fcb = fc_b.reshape(1, 24)

    outT = _tc_lstm(emb3, m0, b0, whh0, wih1, b1, whh1, fcw, fcb)
    return outT.transpose(1, 0, 2).reshape(B, T, 4, 6)


# speedup vs baseline: 86.3207x; 2.0478x over previous
import functools

import jax
import jax.numpy as jnp
from jax import lax
from jax.experimental import pallas as pl
from jax.experimental.pallas import tpu as pltpu
from jax.experimental.pallas import tpu_sc as plsc

B, T = 1024, 200
NUM_CB, CB_SIZE, EMB, HID = 32, 1024, 64, 128
N = B * T
NW = 32
P = N // NW
C = 32
ROWS = C * NUM_CB
NCHUNK = P // C


NBUF = 2


def _sc_embed_body(idx_hbm, tbl_hbm, out_hbm, idx_v, rows_v, out_v, sem):
    wid = lax.axis_index("s") * 2 + lax.axis_index("c")

    def issue(g, b):
        pos0 = pl.multiple_of(wid * P + g * C, C)
        row8 = pl.multiple_of((pos0 * NUM_CB) // 128, 8)
        pltpu.sync_copy(idx_hbm.at[pl.ds(row8, ROWS // 128)], idx_v.at[b])
        for i in range(ROWS // 128):
            pltpu.async_copy(
                tbl_hbm.at[idx_v.at[b].at[i]],
                rows_v.at[b].at[pl.ds(i * 128, 128)],
                sem,
            )

    def wait_chunk(b):
        for i in range(ROWS // 128):
            pltpu.make_async_copy(
                tbl_hbm.at[idx_v.at[b].at[i]],
                rows_v.at[b].at[pl.ds(i * 128, 128)],
                sem,
            ).wait()

    def reduce_store(g, b):
        pos0 = pl.multiple_of(wid * P + g * C, C)

        def pos_body(j, carry2):
            base = j * NUM_CB
            for k in range(EMB // 32):
                sl = pl.ds(k * 32, 32)
                accs = [rows_v[b, base + r, sl] for r in range(4)]
                for r in range(4, NUM_CB):
                    accs[r % 4] = accs[r % 4] + rows_v[b, base + r, sl]
                out_v[j, sl] = (accs[0] + accs[1]) + (accs[2] + accs[3])
            return carry2

        lax.fori_loop(0, C, pos_body, 0, unroll=False)
        pltpu.sync_copy(out_v, out_hbm.at[pl.ds(pos0, C)])

    for b in range(NBUF):
        issue(b, b)

    def body(i, carry):
        for b in range(NBUF):
            g = i * NBUF + b
            wait_chunk(b)
            reduce_store(g, b)

            @pl.when(g + NBUF < NCHUNK)
            def _():
                issue(g + NBUF, b)
        return carry

    lax.fori_loop(0, NCHUNK // NBUF, body, 0, unroll=False)


def _sc_embed(idx2, table):
    mesh = plsc.VectorSubcoreMesh(core_axis_name="c", subcore_axis_name="s")
    return pl.kernel(
        _sc_embed_body,
        mesh=mesh,
        out_type=jax.ShapeDtypeStruct((N, EMB), jnp.bfloat16),
        scratch_types=[
            pltpu.VMEM((NBUF, ROWS // 128, 128), jnp.int32),
            pltpu.VMEM((NBUF, ROWS, EMB), jnp.bfloat16),
            pltpu.VMEM((C, EMB), jnp.bfloat16),
            pltpu.SemaphoreType.DMA,
        ],
        compiler_params=pltpu.CompilerParams(use_tc_tiling_on_sc=False),
    )(idx2, table)


def _tc_body(emb_ref, m0_ref, b0_ref, whh0_ref, wih1_ref, b1_ref, whh1_ref,
             fcw_ref, fcb_ref, out_ref, h0, c0, h1, c1):
    t = pl.program_id(0)

    @pl.when(t == 0)
    def _():
        h0[...] = jnp.zeros_like(h0)
        c0[...] = jnp.zeros_like(c0)
        h1[...] = jnp.zeros_like(h1)
        c1[...] = jnp.zeros_like(c1)

    x = emb_ref[0].astype(jnp.float32)
    g0 = (jnp.dot(x, m0_ref[...], preferred_element_type=jnp.float32)
          + b0_ref[0]
          + jnp.dot(h0[...], whh0_ref[...], preferred_element_type=jnp.float32))
    i0 = jax.nn.sigmoid(g0[:, :HID])
    f0 = jax.nn.sigmoid(g0[:, HID:2 * HID])
    gg0 = jnp.tanh(g0[:, 2 * HID:3 * HID])
    o0 = jax.nn.sigmoid(g0[:, 3 * HID:])
    c0n = f0 * c0[...] + i0 * gg0
    h0n = o0 * jnp.tanh(c0n)
    c0[...] = c0n
    h0[...] = h0n

    g1 = (jnp.dot(h0n, wih1_ref[...], preferred_element_type=jnp.float32)
          + b1_ref[0]
          + jnp.dot(h1[...], whh1_ref[...], preferred_element_type=jnp.float32))
    i1 = jax.nn.sigmoid(g1[:, :HID])
    f1 = jax.nn.sigmoid(g1[:, HID:2 * HID])
    gg1 = jnp.tanh(g1[:, 2 * HID:3 * HID])
    o1 = jax.nn.sigmoid(g1[:, 3 * HID:])
    c1n = f1 * c1[...] + i1 * gg1
    h1n = o1 * jnp.tanh(c1n)
    c1[...] = c1n
    h1[...] = h1n

    out_ref[0] = jnp.dot(h1n, fcw_ref[...], preferred_element_type=jnp.float32) + fcb_ref[0]


def _tc_lstm(emb3, m0, b0, whh0, wih1, b1, whh1, fcw, fcb):
    full = lambda shape: pl.BlockSpec(shape, lambda t: (0,) * len(shape))
    return pl.pallas_call(
        _tc_body,
        grid=(T,),
        in_specs=[
            pl.BlockSpec((1, B, EMB), lambda t: (t, 0, 0)),
            full((EMB, 4 * HID)),
            full((1, 4 * HID)),
            full((HID, 4 * HID)),
            full((HID, 4 * HID)),
            full((1, 4 * HID)),
            full((HID, 4 * HID)),
            full((HID, 24)),
            full((1, 24)),
        ],
        out_specs=pl.BlockSpec((1, B, 24), lambda t: (t, 0, 0)),
        out_shape=jax.ShapeDtypeStruct((T, B, 24), jnp.float32),
        scratch_shapes=[pltpu.VMEM((B, HID), jnp.float32) for _ in range(4)],
        compiler_params=pltpu.CompilerParams(
            dimension_semantics=("arbitrary",)),
    )(emb3, m0, b0, whh0, wih1, b1, whh1, fcw, fcb)


def kernel(x, emb_tables, proj_W, proj_b, l0_W_ih, l0_W_hh, l0_b_ih, l0_b_hh,
           l1_W_ih, l1_W_hh, l1_b_ih, l1_b_hh, fc_W, fc_b):
    offs = (jnp.arange(NUM_CB, dtype=jnp.int32) * CB_SIZE)
    idx = x.transpose(1, 0, 2) + offs
    idx2 = idx.reshape(-1, 128)
    table = emb_tables.reshape(NUM_CB * CB_SIZE, EMB).astype(jnp.bfloat16)

    emb = _sc_embed(idx2, table)
    emb3 = emb.reshape(T, B, EMB)

    m0 = proj_W.T @ l0_W_ih.T
    b0 = (proj_b @ l0_W_ih.T + l0_b_ih + l0_b_hh).reshape(1, 4 * HID)
    whh0 = l0_W_hh.T
    wih1 = l1_W_ih.T
    b1 = (l1_b_ih + l1_b_hh).reshape(1, 4 * HID)
    whh1 = l1_W_hh.T
    fcw = fc_W.reshape(24, HID).T
    fcb = fc_b.reshape(1, 24)

    outT = _tc_lstm(emb3, m0, b0, whh0, wih1, b1, whh1, fcw, fcb)
    return outT.transpose(1, 0, 2).reshape(B, T, 4, 6)


# speedup vs baseline: 93.5188x; 1.0834x over previous
import functools

import jax
import jax.numpy as jnp
from jax import lax
from jax.experimental import pallas as pl
from jax.experimental.pallas import tpu as pltpu
from jax.experimental.pallas import tpu_sc as plsc

B, T = 1024, 200
NUM_CB, CB_SIZE, EMB, HID = 32, 1024, 64, 128
N = B * T
NW = 32
P = N // NW
C = 32
ROWS = C * NUM_CB
NCHUNK = P // C


NBUF = 2
KT = 4
TK = T // KT
NK = B * TK
PK = NK // NW
NCHUNK_K = PK // C


def _sc_embed_body(idx_hbm, tbl_hbm, out_hbm, idx_v, rows_v, out_v, sem):
    wid = lax.axis_index("s") * 2 + lax.axis_index("c")

    def issue(g, b):
        pos0 = pl.multiple_of(wid * PK + g * C, C)
        row8 = pl.multiple_of((pos0 * NUM_CB) // 128, 8)
        pltpu.sync_copy(idx_hbm.at[pl.ds(row8, ROWS // 128)], idx_v.at[b])
        for i in range(ROWS // 128):
            pltpu.async_copy(
                tbl_hbm.at[idx_v.at[b].at[i]],
                rows_v.at[b].at[pl.ds(i * 128, 128)],
                sem,
            )

    def wait_chunk(b):
        for i in range(ROWS // 128):
            pltpu.make_async_copy(
                tbl_hbm.at[idx_v.at[b].at[i]],
                rows_v.at[b].at[pl.ds(i * 128, 128)],
                sem,
            ).wait()

    def reduce_store(g, b):
        pos0 = pl.multiple_of(wid * PK + g * C, C)

        def pos_body(j, carry2):
            base = j * NUM_CB
            for k in range(EMB // 32):
                sl = pl.ds(k * 32, 32)
                accs = [rows_v[b, base + r, sl] for r in range(4)]
                for r in range(4, NUM_CB):
                    accs[r % 4] = accs[r % 4] + rows_v[b, base + r, sl]
                out_v[j, sl] = (accs[0] + accs[1]) + (accs[2] + accs[3])
            return carry2

        lax.fori_loop(0, C, pos_body, 0, unroll=False)
        pltpu.sync_copy(out_v, out_hbm.at[pl.ds(pos0, C)])

    for b in range(NBUF):
        issue(b, b)

    def body(i, carry):
        for b in range(NBUF):
            g = i * NBUF + b
            wait_chunk(b)
            reduce_store(g, b)

            @pl.when(g + NBUF < NCHUNK_K)
            def _():
                issue(g + NBUF, b)
        return carry

    lax.fori_loop(0, NCHUNK_K // NBUF, body, 0, unroll=False)


def _sc_embed(idx2, table):
    mesh = plsc.VectorSubcoreMesh(core_axis_name="c", subcore_axis_name="s")
    return pl.kernel(
        _sc_embed_body,
        mesh=mesh,
        out_type=jax.ShapeDtypeStruct((NK, EMB), jnp.bfloat16),
        scratch_types=[
            pltpu.VMEM((NBUF, ROWS // 128, 128), jnp.int32),
            pltpu.VMEM((NBUF, ROWS, EMB), jnp.bfloat16),
            pltpu.VMEM((C, EMB), jnp.bfloat16),
            pltpu.SemaphoreType.DMA,
        ],
        compiler_params=pltpu.CompilerParams(use_tc_tiling_on_sc=False),
    )(idx2, table)


def _tc_body(emb_ref, m0_ref, b0_ref, whh0_ref, wih1_ref, b1_ref, whh1_ref,
             fcw_ref, fcb_ref, h0i, c0i, h1i, c1i,
             out_ref, h0, c0, h1, c1):
    t = pl.program_id(0)

    @pl.when(t == 0)
    def _():
        h0[...] = h0i[...]
        c0[...] = c0i[...]
        h1[...] = h1i[...]
        c1[...] = c1i[...]

    x = emb_ref[0].astype(jnp.float32)
    g0 = (jnp.dot(x, m0_ref[...], preferred_element_type=jnp.float32)
          + b0_ref[0]
          + jnp.dot(h0[...], whh0_ref[...], preferred_element_type=jnp.float32))
    i0 = jax.nn.sigmoid(g0[:, :HID])
    f0 = jax.nn.sigmoid(g0[:, HID:2 * HID])
    gg0 = jnp.tanh(g0[:, 2 * HID:3 * HID])
    o0 = jax.nn.sigmoid(g0[:, 3 * HID:])
    c0n = f0 * c0[...] + i0 * gg0
    h0n = o0 * jnp.tanh(c0n)
    c0[...] = c0n
    h0[...] = h0n

    g1 = (jnp.dot(h0n, wih1_ref[...], preferred_element_type=jnp.float32)
          + b1_ref[0]
          + jnp.dot(h1[...], whh1_ref[...], preferred_element_type=jnp.float32))
    i1 = jax.nn.sigmoid(g1[:, :HID])
    f1 = jax.nn.sigmoid(g1[:, HID:2 * HID])
    gg1 = jnp.tanh(g1[:, 2 * HID:3 * HID])
    o1 = jax.nn.sigmoid(g1[:, 3 * HID:])
    c1n = f1 * c1[...] + i1 * gg1
    h1n = o1 * jnp.tanh(c1n)
    c1[...] = c1n
    h1[...] = h1n

    out_ref[0] = jnp.dot(h1n, fcw_ref[...], preferred_element_type=jnp.float32) + fcb_ref[0]


def _tc_lstm(emb3, m0, b0, whh0, wih1, b1, whh1, fcw, fcb, h0, c0, h1, c1):
    full = lambda shape: pl.BlockSpec(shape, lambda t: (0,) * len(shape))
    state_spec = full((B, HID))
    state_shape = jax.ShapeDtypeStruct((B, HID), jnp.float32)
    return pl.pallas_call(
        _tc_body,
        grid=(TK,),
        in_specs=[
            pl.BlockSpec((1, B, EMB), lambda t: (t, 0, 0)),
            full((EMB, 4 * HID)),
            full((1, 4 * HID)),
            full((HID, 4 * HID)),
            full((HID, 4 * HID)),
            full((1, 4 * HID)),
            full((HID, 4 * HID)),
            full((HID, 24)),
            full((1, 24)),
            state_spec, state_spec, state_spec, state_spec,
        ],
        out_specs=[
            pl.BlockSpec((1, B, 24), lambda t: (t, 0, 0)),
            state_spec, state_spec, state_spec, state_spec,
        ],
        out_shape=[
            jax.ShapeDtypeStruct((TK, B, 24), jnp.float32),
            state_shape, state_shape, state_shape, state_shape,
        ],
        compiler_params=pltpu.CompilerParams(
            dimension_semantics=("arbitrary",)),
    )(emb3, m0, b0, whh0, wih1, b1, whh1, fcw, fcb, h0, c0, h1, c1)


def kernel(x, emb_tables, proj_W, proj_b, l0_W_ih, l0_W_hh, l0_b_ih, l0_b_hh,
           l1_W_ih, l1_W_hh, l1_b_ih, l1_b_hh, fc_W, fc_b):
    offs = (jnp.arange(NUM_CB, dtype=jnp.int32) * CB_SIZE)
    idx = x.transpose(1, 0, 2) + offs
    idx2 = idx.reshape(-1, 128)
    table = emb_tables.reshape(NUM_CB * CB_SIZE, EMB).astype(jnp.bfloat16)

    m0 = proj_W.T @ l0_W_ih.T
    b0 = (proj_b @ l0_W_ih.T + l0_b_ih + l0_b_hh).reshape(1, 4 * HID)
    whh0 = l0_W_hh.T
    wih1 = l1_W_ih.T
    b1 = (l1_b_ih + l1_b_hh).reshape(1, 4 * HID)
    whh1 = l1_W_hh.T
    fcw = fc_W.reshape(24, HID).T
    fcb = fc_b.reshape(1, 24)

    rows_per_k = NK * NUM_CB // 128
    state = [jnp.zeros((B, HID), jnp.float32) for _ in range(4)]
    outs = []
    for k in range(KT):
        embk = _sc_embed(idx2[k * rows_per_k:(k + 1) * rows_per_k], table)
        outk, *state = _tc_lstm(embk.reshape(TK, B, EMB), m0, b0, whh0,
                                wih1, b1, whh1, fcw, fcb, *state)
        outs.append(outk)
    outT = jnp.concatenate(outs, axis=0)
    return outT.transpose(1, 0, 2).reshape(B, T, 4, 6)


# speedup vs baseline: 93.6053x; 1.0009x over previous
import functools

import jax
import jax.numpy as jnp
from jax import lax
from jax.experimental import pallas as pl
from jax.experimental.pallas import tpu as pltpu
from jax.experimental.pallas import tpu_sc as plsc

B, T = 1024, 200
NUM_CB, CB_SIZE, EMB, HID = 32, 1024, 64, 128
N = B * T
NW = 32
P = N // NW
C = 32
ROWS = C * NUM_CB
NCHUNK = P // C


NBUF = 2
KT = 4
TK = T // KT
NK = B * TK
PK = NK // NW
NCHUNK_K = PK // C


def _sc_embed_body(idx_hbm, tbl_hbm, out_hbm, idx_v, rows_v, out_v, sem):
    wid = lax.axis_index("s") * 2 + lax.axis_index("c")

    def issue(g, b):
        pos0 = pl.multiple_of(wid * PK + g * C, C)
        row8 = pl.multiple_of((pos0 * NUM_CB) // 128, 8)
        pltpu.sync_copy(idx_hbm.at[pl.ds(row8, ROWS // 128)], idx_v.at[b])
        for i in range(ROWS // 128):
            pltpu.async_copy(
                tbl_hbm.at[idx_v.at[b].at[i]],
                rows_v.at[b].at[pl.ds(i * 128, 128)],
                sem,
            )

    def wait_chunk(b):
        for i in range(ROWS // 128):
            pltpu.make_async_copy(
                tbl_hbm.at[idx_v.at[b].at[i]],
                rows_v.at[b].at[pl.ds(i * 128, 128)],
                sem,
            ).wait()

    def reduce_store(g, b):
        pos0 = pl.multiple_of(wid * PK + g * C, C)

        def pos_body(j, carry2):
            base = j * NUM_CB
            for k in range(EMB // 32):
                sl = pl.ds(k * 32, 32)
                accs = [rows_v[b, base + r, sl] for r in range(4)]
                for r in range(4, NUM_CB):
                    accs[r % 4] = accs[r % 4] + rows_v[b, base + r, sl]
                out_v[j, sl] = (accs[0] + accs[1]) + (accs[2] + accs[3])
            return carry2

        lax.fori_loop(0, C, pos_body, 0, unroll=False)
        pltpu.sync_copy(out_v, out_hbm.at[pl.ds(pos0, C)])

    for b in range(NBUF):
        issue(b, b)

    def body(i, carry):
        for b in range(NBUF):
            g = i * NBUF + b
            wait_chunk(b)
            reduce_store(g, b)

            @pl.when(g + NBUF < NCHUNK_K)
            def _():
                issue(g + NBUF, b)
        return carry

    lax.fori_loop(0, NCHUNK_K // NBUF, body, 0, unroll=False)


def _sc_embed(idx2, table):
    mesh = plsc.VectorSubcoreMesh(core_axis_name="c", subcore_axis_name="s")
    return pl.kernel(
        _sc_embed_body,
        mesh=mesh,
        out_type=jax.ShapeDtypeStruct((NK, EMB), jnp.bfloat16),
        scratch_types=[
            pltpu.VMEM((NBUF, ROWS // 128, 128), jnp.int32),
            pltpu.VMEM((NBUF, ROWS, EMB), jnp.bfloat16),
            pltpu.VMEM((C, EMB), jnp.bfloat16),
            pltpu.SemaphoreType.DMA,
        ],
        compiler_params=pltpu.CompilerParams(use_tc_tiling_on_sc=False),
    )(idx2, table)


def _tc_body(emb_ref, m0_ref, b0_ref, whh0_ref, wih1_ref, b1_ref, whh1_ref,
             fcw_ref, fcb_ref, h0i, c0i, h1i, c1i,
             out_ref, h0, c0, h1, c1):
    t = pl.program_id(0)

    @pl.when(t == 0)
    def _():
        h0[...] = h0i[...]
        c0[...] = c0i[...]
        h1[...] = h1i[...]
        c1[...] = c1i[...]

    x = emb_ref[0].astype(jnp.float32)
    g0 = (jnp.dot(x, m0_ref[...], preferred_element_type=jnp.float32)
          + b0_ref[0]
          + jnp.dot(h0[...], whh0_ref[...], preferred_element_type=jnp.float32))
    i0 = jax.nn.sigmoid(g0[:, :HID])
    f0 = jax.nn.sigmoid(g0[:, HID:2 * HID])
    gg0 = jnp.tanh(g0[:, 2 * HID:3 * HID])
    o0 = jax.nn.sigmoid(g0[:, 3 * HID:])
    c0n = f0 * c0[...] + i0 * gg0
    h0n = o0 * jnp.tanh(c0n)
    c0[...] = c0n
    h0[...] = h0n

    g1 = (jnp.dot(h0n, wih1_ref[...], preferred_element_type=jnp.float32)
          + b1_ref[0]
          + jnp.dot(h1[...], whh1_ref[...], preferred_element_type=jnp.float32))
    i1 = jax.nn.sigmoid(g1[:, :HID])
    f1 = jax.nn.sigmoid(g1[:, HID:2 * HID])
    gg1 = jnp.tanh(g1[:, 2 * HID:3 * HID])
    o1 = jax.nn.sigmoid(g1[:, 3 * HID:])
    c1n = f1 * c1[...] + i1 * gg1
    h1n = o1 * jnp.tanh(c1n)
    c1[...] = c1n
    h1[...] = h1n

    out_ref[0] = jnp.dot(h1n, fcw_ref[...], preferred_element_type=jnp.float32) + fcb_ref[0]


def _tc_lstm(emb3, m0, b0, whh0, wih1, b1, whh1, fcw, fcb, h0, c0, h1, c1):
    full = lambda shape: pl.BlockSpec(shape, lambda t: (0,) * len(shape))
    state_spec = full((B, HID))
    state_shape = jax.ShapeDtypeStruct((B, HID), jnp.float32)
    return pl.pallas_call(
        _tc_body,
        grid=(TK,),
        in_specs=[
            pl.BlockSpec((1, B, EMB), lambda t: (t, 0, 0)),
            full((EMB, 4 * HID)),
            full((1, 4 * HID)),
            full((HID, 4 * HID)),
            full((HID, 4 * HID)),
            full((1, 4 * HID)),
            full((HID, 4 * HID)),
            full((HID, 24)),
            full((1, 24)),
            state_spec, state_spec, state_spec, state_spec,
        ],
        out_specs=[
            pl.BlockSpec((1, B, 24), lambda t: (t, 0, 0)),
            state_spec, state_spec, state_spec, state_spec,
        ],
        out_shape=[
            jax.ShapeDtypeStruct((TK, B, 24), jnp.float32),
            state_shape, state_shape, state_shape, state_shape,
        ],
        compiler_params=pltpu.CompilerParams(
            dimension_semantics=("arbitrary",)),
    )(emb3, m0, b0, whh0, wih1, b1, whh1, fcw, fcb, h0, c0, h1, c1)


def kernel(x, emb_tables, proj_W, proj_b, l0_W_ih, l0_W_hh, l0_b_ih, l0_b_hh,
           l1_W_ih, l1_W_hh, l1_b_ih, l1_b_hh, fc_W, fc_b):
    offs = (jnp.arange(NUM_CB, dtype=jnp.int32) * CB_SIZE)
    idx = x.transpose(1, 0, 2) + offs
    idx2 = idx.reshape(-1, 128)
    table = emb_tables.reshape(NUM_CB * CB_SIZE, EMB).astype(jnp.bfloat16)

    m0 = proj_W.T @ l0_W_ih.T
    b0 = (proj_b @ l0_W_ih.T + l0_b_ih + l0_b_hh).reshape(1, 4 * HID)
    whh0 = l0_W_hh.T
    wih1 = l1_W_ih.T
    b1 = (l1_b_ih + l1_b_hh).reshape(1, 4 * HID)
    whh1 = l1_W_hh.T
    fcw = fc_W.reshape(24, HID).T
    fcb = fc_b.reshape(1, 24)

    rows_per_k = NK * NUM_CB // 128
    embs = [_sc_embed(idx2[k * rows_per_k:(k + 1) * rows_per_k], table)
            for k in range(KT)]
    state = [jnp.zeros((B, HID), jnp.float32) for _ in range(4)]
    outs = []
    for k in range(KT):
        outk, *state = _tc_lstm(embs[k].reshape(TK, B, EMB), m0, b0, whh0,
                                wih1, b1, whh1, fcw, fcb, *state)
        outs.append(outk)
    outT = jnp.concatenate(outs, axis=0)
    return outT.transpose(1, 0, 2).reshape(B, T, 4, 6)


# speedup vs baseline: 96.4670x; 1.0306x over previous
import functools

import jax
import jax.numpy as jnp
from jax import lax
from jax.experimental import pallas as pl
from jax.experimental.pallas import tpu as pltpu
from jax.experimental.pallas import tpu_sc as plsc

B, T = 1024, 200
NUM_CB, CB_SIZE, EMB, HID = 32, 1024, 64, 128
N = B * T
NW = 32
P = N // NW
C = 32
ROWS = C * NUM_CB
NCHUNK = P // C


NBUF = 2
KT = 4
TK = T // KT
NK = B * TK
PK = NK // NW
NCHUNK_K = PK // C


def _sc_embed_body(idx_hbm, tbl_hbm, out_hbm, idx_v, rows_v, out_v, sem):
    wid = lax.axis_index("s") * 2 + lax.axis_index("c")

    def issue(g, b):
        pos0 = pl.multiple_of(wid * PK + g * C, C)
        row8 = pl.multiple_of((pos0 * NUM_CB) // 128, 8)
        pltpu.sync_copy(idx_hbm.at[pl.ds(row8, ROWS // 128)], idx_v.at[b])
        for i in range(ROWS // 128):
            pltpu.async_copy(
                tbl_hbm.at[idx_v.at[b].at[i]],
                rows_v.at[b].at[pl.ds(i * 128, 128)],
                sem,
            )

    def wait_chunk(b):
        for i in range(ROWS // 128):
            pltpu.make_async_copy(
                tbl_hbm.at[idx_v.at[b].at[i]],
                rows_v.at[b].at[pl.ds(i * 128, 128)],
                sem,
            ).wait()

    def reduce_store(g, b):
        pos0 = pl.multiple_of(wid * PK + g * C, C)

        def pos_body(j, carry2):
            base = j * NUM_CB
            for k in range(EMB // 32):
                sl = pl.ds(k * 32, 32)
                accs = [rows_v[b, base + r, sl] for r in range(4)]
                for r in range(4, NUM_CB):
                    accs[r % 4] = accs[r % 4] + rows_v[b, base + r, sl]
                out_v[j, sl] = (accs[0] + accs[1]) + (accs[2] + accs[3])
            return carry2

        lax.fori_loop(0, C, pos_body, 0, unroll=False)
        pltpu.sync_copy(out_v, out_hbm.at[pl.ds(pos0, C)])

    for b in range(NBUF):
        issue(b, b)

    def body(i, carry):
        for b in range(NBUF):
            g = i * NBUF + b
            wait_chunk(b)
            reduce_store(g, b)

            @pl.when(g + NBUF < NCHUNK_K)
            def _():
                issue(g + NBUF, b)
        return carry

    lax.fori_loop(0, NCHUNK_K // NBUF, body, 0, unroll=False)


def _sc_embed(idx2, table):
    mesh = plsc.VectorSubcoreMesh(core_axis_name="c", subcore_axis_name="s")
    return pl.kernel(
        _sc_embed_body,
        mesh=mesh,
        out_type=jax.ShapeDtypeStruct((NK, EMB), jnp.bfloat16),
        scratch_types=[
            pltpu.VMEM((NBUF, ROWS // 128, 128), jnp.int32),
            pltpu.VMEM((NBUF, ROWS, EMB), jnp.bfloat16),
            pltpu.VMEM((C, EMB), jnp.bfloat16),
            pltpu.SemaphoreType.DMA,
        ],
        compiler_params=pltpu.CompilerParams(use_tc_tiling_on_sc=False),
    )(idx2, table)


def _tc_body(emb_ref, m0_ref, b0_ref, whh0_ref, wih1_ref, b1_ref, whh1_ref,
             fcw_ref, fcb_ref, h0i, c0i, h1i, c1i,
             out_ref, h0, c0, h1, c1):
    t = pl.program_id(0)

    @pl.when(t == 0)
    def _():
        h0[...] = h0i[...]
        c0[...] = c0i[...]
        h1[...] = h1i[...]
        c1[...] = c1i[...]

    def sig(v):
        return 0.5 * jnp.tanh(0.5 * v) + 0.5

    x = emb_ref[0]
    g0 = (jnp.dot(x, m0_ref[...], preferred_element_type=jnp.float32)
          + b0_ref[0]
          + jnp.dot(h0[...].astype(jnp.bfloat16), whh0_ref[...],
                    preferred_element_type=jnp.float32))
    i0 = sig(g0[:, :HID])
    f0 = sig(g0[:, HID:2 * HID])
    gg0 = jnp.tanh(g0[:, 2 * HID:3 * HID])
    o0 = sig(g0[:, 3 * HID:])
    c0n = f0 * c0[...] + i0 * gg0
    h0n = o0 * jnp.tanh(c0n)
    c0[...] = c0n
    h0[...] = h0n

    h0b = h0n.astype(jnp.bfloat16)
    g1 = (jnp.dot(h0b, wih1_ref[...], preferred_element_type=jnp.float32)
          + b1_ref[0]
          + jnp.dot(h1[...].astype(jnp.bfloat16), whh1_ref[...],
                    preferred_element_type=jnp.float32))
    i1 = sig(g1[:, :HID])
    f1 = sig(g1[:, HID:2 * HID])
    gg1 = jnp.tanh(g1[:, 2 * HID:3 * HID])
    o1 = sig(g1[:, 3 * HID:])
    c1n = f1 * c1[...] + i1 * gg1
    h1n = o1 * jnp.tanh(c1n)
    c1[...] = c1n
    h1[...] = h1n

    out_ref[0] = (jnp.dot(h1n.astype(jnp.bfloat16), fcw_ref[...],
                          preferred_element_type=jnp.float32) + fcb_ref[0])


def _tc_lstm(emb3, m0, b0, whh0, wih1, b1, whh1, fcw, fcb, h0, c0, h1, c1):
    full = lambda shape: pl.BlockSpec(shape, lambda t: (0,) * len(shape))
    state_spec = full((B, HID))
    state_shape = jax.ShapeDtypeStruct((B, HID), jnp.float32)
    return pl.pallas_call(
        _tc_body,
        grid=(TK,),
        in_specs=[
            pl.BlockSpec((1, B, EMB), lambda t: (t, 0, 0)),
            full((EMB, 4 * HID)),
            full((1, 4 * HID)),
            full((HID, 4 * HID)),
            full((HID, 4 * HID)),
            full((1, 4 * HID)),
            full((HID, 4 * HID)),
            full((HID, 24)),
            full((1, 24)),
            state_spec, state_spec, state_spec, state_spec,
        ],
        out_specs=[
            pl.BlockSpec((1, B, 24), lambda t: (t, 0, 0)),
            state_spec, state_spec, state_spec, state_spec,
        ],
        out_shape=[
            jax.ShapeDtypeStruct((TK, B, 24), jnp.float32),
            state_shape, state_shape, state_shape, state_shape,
        ],
        compiler_params=pltpu.CompilerParams(
            dimension_semantics=("arbitrary",)),
    )(emb3, m0, b0, whh0, wih1, b1, whh1, fcw, fcb, h0, c0, h1, c1)


def kernel(x, emb_tables, proj_W, proj_b, l0_W_ih, l0_W_hh, l0_b_ih, l0_b_hh,
           l1_W_ih, l1_W_hh, l1_b_ih, l1_b_hh, fc_W, fc_b):
    offs = (jnp.arange(NUM_CB, dtype=jnp.int32) * CB_SIZE)
    idx = x.transpose(1, 0, 2) + offs
    idx2 = idx.reshape(-1, 128)
    table = emb_tables.reshape(NUM_CB * CB_SIZE, EMB).astype(jnp.bfloat16)

    m0 = (proj_W.T @ l0_W_ih.T).astype(jnp.bfloat16)
    b0 = (proj_b @ l0_W_ih.T + l0_b_ih + l0_b_hh).reshape(1, 4 * HID)
    whh0 = l0_W_hh.T.astype(jnp.bfloat16)
    wih1 = l1_W_ih.T.astype(jnp.bfloat16)
    b1 = (l1_b_ih + l1_b_hh).reshape(1, 4 * HID)
    whh1 = l1_W_hh.T.astype(jnp.bfloat16)
    fcw = fc_W.reshape(24, HID).T.astype(jnp.bfloat16)
    fcb = fc_b.reshape(1, 24)

    rows_per_k = NK * NUM_CB // 128
    embs = [_sc_embed(idx2[k * rows_per_k:(k + 1) * rows_per_k], table)
            for k in range(KT)]
    state = [jnp.zeros((B, HID), jnp.float32) for _ in range(4)]
    outs = []
    for k in range(KT):
        outk, *state = _tc_lstm(embs[k].reshape(TK, B, EMB), m0, b0, whh0,
                                wih1, b1, whh1, fcw, fcb, *state)
        outs.append(outk)
    outT = jnp.concatenate(outs, axis=0)
    return outT.transpose(1, 0, 2).reshape(B, T, 4, 6)
